# baseline (speedup 1.0000x reference)
import functools
import math

import jax
import jax.numpy as jnp
from jax import lax
from jax.experimental import pallas as pl
from jax.experimental.pallas import tpu as pltpu
from jax.experimental.pallas import tpu_sc as plsc

D_MODEL = 64
SEQ = 50
SCALE = math.sqrt(D_MODEL)

_info = plsc.get_sparse_core_info()
NC = _info.num_cores
NS = _info.num_subcores
NW = NC * NS

CHUNK_BATCH = 4
CHUNK_ROWS = CHUNK_BATCH * SEQ
SUBS = ((0, 104), (104, 96))


def _emb(idx_flat, table, pe2d, n_rows):
    rows_per_w = n_rows // NW
    n_chunks = rows_per_w // CHUNK_ROWS
    mesh = plsc.VectorSubcoreMesh(core_axis_name="c", subcore_axis_name="s")

    @functools.partial(
        pl.kernel,
        mesh=mesh,
        out_type=jax.ShapeDtypeStruct((n_rows, D_MODEL), jnp.float32),
        scratch_types=[
            pltpu.VMEM((CHUNK_ROWS,), jnp.int32),
            pltpu.VMEM((CHUNK_ROWS, D_MODEL), jnp.float32),
            pltpu.VMEM((SEQ, D_MODEL), jnp.float32),
            pltpu.SemaphoreType.DMA,
        ],
        compiler_params=pltpu.CompilerParams(use_tc_tiling_on_sc=False),
    )
    def body(idx_hbm, table_hbm, pe_hbm, out_hbm, idx_v, buf_v, pe_v, sem):
        wid = lax.axis_index("s") * NC + lax.axis_index("c")
        wbase = wid * rows_per_w
        pltpu.sync_copy(pe_hbm, pe_v)

        def chunk_body(c, carry):
            gbase = pl.multiple_of(wbase + c * CHUNK_ROWS, CHUNK_ROWS)
            pltpu.sync_copy(idx_hbm.at[pl.ds(gbase, CHUNK_ROWS)], idx_v)
            cps = [
                pltpu.async_copy(
                    table_hbm.at[idx_v.at[pl.ds(off, sz)]],
                    buf_v.at[pl.ds(off, sz)],
                    sem,
                )
                for off, sz in SUBS
            ]
            for cp in cps:
                cp.wait()

            def pos_body(s, carry2):
                for q in range(D_MODEL // 16):
                    peq = pe_v[s, pl.ds(q * 16, 16)]
                    for b in range(CHUNK_BATCH):
                        r = b * SEQ + s
                        buf_v[r, pl.ds(q * 16, 16)] = (
                            buf_v[r, pl.ds(q * 16, 16)] * SCALE + peq
                        )
                return carry2

            lax.fori_loop(0, SEQ, pos_body, 0)
            pltpu.sync_copy(buf_v, out_hbm.at[pl.ds(gbase, CHUNK_ROWS)])
            return carry

        lax.fori_loop(0, n_chunks, chunk_body, 0)

    return body(idx_flat, table, pe2d)


def kernel(encoded_words, table, pe):
    batch, seq = encoded_words.shape
    n_rows = batch * seq
    idx_flat = encoded_words.reshape(n_rows)
    pe2d = pe.reshape(pe.shape[1], pe.shape[2])[:seq]
    out = _emb(idx_flat, table, pe2d, n_rows)
    return out.reshape(batch, seq, D_MODEL)


# speedup vs baseline: 1.0626x; 1.0626x over previous
import functools
import math

import jax
import jax.numpy as jnp
from jax import lax
from jax.experimental import pallas as pl
from jax.experimental.pallas import tpu as pltpu
from jax.experimental.pallas import tpu_sc as plsc

D_MODEL = 64
SEQ = 50
SCALE = math.sqrt(D_MODEL)

_info = plsc.get_sparse_core_info()
NC = _info.num_cores
NS = _info.num_subcores
NW = NC * NS

CHUNK_BATCH = 4
CHUNK_ROWS = CHUNK_BATCH * SEQ
SUBS = ((0, 104), (104, 96))
NBUF = 4


def _emb(idx_flat, table, pe2d, n_rows):
    rows_per_w = n_rows // NW
    n_chunks = rows_per_w // CHUNK_ROWS
    mesh = plsc.VectorSubcoreMesh(core_axis_name="c", subcore_axis_name="s")

    @functools.partial(
        pl.kernel,
        mesh=mesh,
        out_type=jax.ShapeDtypeStruct((n_rows, D_MODEL), jnp.float32),
        scratch_types=[
            pltpu.VMEM((rows_per_w,), jnp.int32),
            [pltpu.VMEM((CHUNK_ROWS, D_MODEL), jnp.float32)] * NBUF,
            [pltpu.VMEM((CHUNK_ROWS, D_MODEL), jnp.float32)] * NBUF,
            pltpu.VMEM((SEQ, D_MODEL), jnp.float32),
            [pltpu.SemaphoreType.DMA] * NBUF,
            [pltpu.SemaphoreType.DMA] * NBUF,
        ],
        compiler_params=pltpu.CompilerParams(use_tc_tiling_on_sc=False),
    )
    def body(idx_hbm, table_hbm, pe_hbm, out_hbm, idx_all, gbufs, obufs,
             pe_v, gsems, wsems):
        wid = lax.axis_index("s") * NC + lax.axis_index("c")
        wbase = pl.multiple_of(wid * rows_per_w, rows_per_w)
        pltpu.sync_copy(pe_hbm, pe_v)
        pltpu.sync_copy(idx_hbm.at[pl.ds(wbase, rows_per_w)], idx_all)

        def start_gather(c, b):
            return [
                pltpu.async_copy(
                    table_hbm.at[idx_all.at[pl.ds(c * CHUNK_ROWS + off, sz)]],
                    gbufs[b].at[pl.ds(off, sz)],
                    gsems[b],
                )
                for off, sz in SUBS
            ]

        def compute(b):
            def pos_body(s, carry):
                for q in range(D_MODEL // 16):
                    peq = pe_v[s, pl.ds(q * 16, 16)]
                    for bb in range(CHUNK_BATCH):
                        r = bb * SEQ + s
                        obufs[b][r, pl.ds(q * 16, 16)] = (
                            gbufs[b][r, pl.ds(q * 16, 16)] * SCALE + peq
                        )
                return carry

            lax.fori_loop(0, SEQ, pos_body, 0)

        gcp = [start_gather(b, b) for b in range(NBUF)]
        wcp = [None] * NBUF
        for c in range(n_chunks):
            b = c % NBUF
            for cp in gcp[b]:
                cp.wait()
            if wcp[b] is not None:
                wcp[b].wait()
            compute(b)
            if c + NBUF < n_chunks:
                gcp[b] = start_gather(c + NBUF, b)
            wcp[b] = pltpu.async_copy(
                obufs[b],
                out_hbm.at[pl.ds(wbase + c * CHUNK_ROWS, CHUNK_ROWS)],
                wsems[b],
            )
        for b in range(NBUF):
            wcp[b].wait()

    return body(idx_flat, table, pe2d)


def kernel(encoded_words, table, pe):
    batch, seq = encoded_words.shape
    n_rows = batch * seq
    idx_flat = encoded_words.reshape(n_rows)
    pe2d = pe.reshape(pe.shape[1], pe.shape[2])[:seq]
    out = _emb(idx_flat, table, pe2d, n_rows)
    return out.reshape(batch, seq, D_MODEL)


# speedup vs baseline: 1.4592x; 1.3732x over previous
import functools
import math

import jax
import jax.numpy as jnp
from jax import lax
from jax.experimental import pallas as pl
from jax.experimental.pallas import tpu as pltpu
from jax.experimental.pallas import tpu_sc as plsc

D_MODEL = 64
SEQ = 50
SCALE = math.sqrt(D_MODEL)

_info = plsc.get_sparse_core_info()
NC = _info.num_cores
NS = _info.num_subcores
NW = NC * NS

CHUNK_BATCH = 4
CHUNK_ROWS = CHUNK_BATCH * SEQ
SUBS = ((0, 128), (128, 72))
TBLK = 8192


def _widen_transpose(table_t, vocab):
    grid = (vocab + TBLK - 1) // TBLK

    def tbody(x_ref, o_ref):
        o_ref[:, 0:D_MODEL] = jnp.transpose(x_ref[...])

    return pl.pallas_call(
        tbody,
        grid=(grid,),
        in_specs=[pl.BlockSpec((D_MODEL, TBLK), lambda i: (0, i))],
        out_specs=pl.BlockSpec((TBLK, 128), lambda i: (i, 0)),
        out_shape=jax.ShapeDtypeStruct((vocab, 128), jnp.float32),
    )(table_t)


def _gather_pe(idx_flat, wide, pe_flat, n_rows):
    rows_per_w = n_rows // NW
    n_chunks = rows_per_w // CHUNK_ROWS
    pk_per_w = rows_per_w // 2
    mesh = plsc.VectorSubcoreMesh(core_axis_name="c", subcore_axis_name="s")

    @functools.partial(
        pl.kernel,
        mesh=mesh,
        out_type=jax.ShapeDtypeStruct((n_rows // 2, 128), jnp.float32),
        scratch_types=[
            pltpu.VMEM((rows_per_w,), jnp.int32),
            [pltpu.VMEM((CHUNK_ROWS, 128), jnp.float32)] * 2,
            [pltpu.VMEM((CHUNK_ROWS, 128), jnp.float32)] * 2,
            pltpu.VMEM((SEQ * D_MODEL,), jnp.float32),
            [pltpu.SemaphoreType.DMA] * 2,
            [pltpu.SemaphoreType.DMA] * 2,
        ],
        compiler_params=pltpu.CompilerParams(use_tc_tiling_on_sc=True),
    )
    def body(idx_hbm, wide_hbm, pe_hbm, out_hbm, idx_all, gbufs, obufs,
             pe_v, gsems, wsems):
        wid = lax.axis_index("s") * NC + lax.axis_index("c")
        wbase = pl.multiple_of(wid * rows_per_w, rows_per_w)
        wpk = pl.multiple_of(wid * pk_per_w, pk_per_w)
        pltpu.sync_copy(pe_hbm, pe_v)
        pltpu.sync_copy(idx_hbm.at[pl.ds(wbase, rows_per_w)], idx_all)

        def start_gather(c, gb):
            for off, sz in SUBS:
                pltpu.async_copy(
                    wide_hbm.at[
                        idx_all.at[
                            pl.ds(pl.multiple_of(c * CHUNK_ROWS + off, 8), sz)
                        ]
                    ],
                    gbufs[gb].at[pl.ds(off, sz)],
                    gsems[gb],
                )

        def compute(gb, ob, half):
            def sp_body(sp, carry):
                pbase = pl.multiple_of(sp * 2 * D_MODEL, 8)
                pe_lo = [pe_v[pl.ds(pbase + q * 16, 16)] for q in range(4)]
                pe_hi = [pe_v[pl.ds(pbase + D_MODEL + q * 16, 16)]
                         for q in range(4)]
                for bb in range(CHUNK_BATCH):
                    r_pk = half * (CHUNK_ROWS // 2) + bb * (SEQ // 2) + sp
                    g0 = bb * SEQ + 2 * sp
                    for q in range(4):
                        obufs[ob][r_pk, pl.ds(q * 16, 16)] = (
                            gbufs[gb][g0, pl.ds(q * 16, 16)] * SCALE
                            + pe_lo[q]
                        )
                        obufs[ob][r_pk, pl.ds(D_MODEL + q * 16, 16)] = (
                            gbufs[gb][g0 + 1, pl.ds(q * 16, 16)] * SCALE
                            + pe_hi[q]
                        )
                return carry

            lax.fori_loop(0, SEQ // 2, sp_body, 0)

        def wait_gather(gb):
            for off, sz in SUBS:
                pltpu.make_async_copy(
                    wide_hbm.at[idx_all.at[pl.ds(off, sz)]],
                    gbufs[gb].at[pl.ds(off, sz)],
                    gsems[gb],
                ).wait()

        def wait_wb(ob):
            pltpu.make_async_copy(
                obufs[ob],
                out_hbm.at[pl.ds(wpk, CHUNK_ROWS)],
                wsems[ob],
            ).wait()

        start_gather(0, 0)
        start_gather(1, 1)

        def group_body(g, carry):
            for dc in range(4):
                c = g * 4 + dc
                gb = dc % 2
                ob = (dc // 2) % 2
                half = dc % 2
                wait_gather(gb)
                if half == 0:
                    @pl.when(g > 0)
                    def _():
                        wait_wb(ob)
                compute(gb, ob, half)

                @pl.when(c + 2 < n_chunks)
                def _():
                    start_gather(c + 2, gb)

                if half == 1:
                    pltpu.async_copy(
                        obufs[ob],
                        out_hbm.at[pl.ds(
                            wpk + (g * 2 + dc // 2) * CHUNK_ROWS, CHUNK_ROWS)],
                        wsems[ob],
                    )
            return carry

        lax.fori_loop(0, n_chunks // 4, group_body, 0)
        for ob in range(2):
            wait_wb(ob)

    return body(idx_flat, wide, pe_flat)


def kernel(encoded_words, table, pe):
    batch, seq = encoded_words.shape
    vocab = table.shape[0]
    n_rows = batch * seq
    idx_flat = encoded_words.reshape(n_rows)
    pe_flat = pe.reshape(pe.shape[1] * pe.shape[2])[: seq * D_MODEL]
    wide = _widen_transpose(table.T, vocab)
    out_pk = _gather_pe(idx_flat, wide, pe_flat, n_rows)
    return out_pk.reshape(n_rows, D_MODEL).reshape(batch, seq, D_MODEL)
